# baseline (speedup 1.0000x reference)
import jax
import jax.numpy as jnp
from jax.experimental import pallas as pl


def kernel(logits, boxes, original_sizes):
    cxcy = boxes[..., :2]
    wh = boxes[..., 2:]
    boxes_xy = jnp.concatenate([cxcy - wh * 0.5, wh], axis=-1)
    sizes_wh = jnp.stack([original_sizes[0][1], original_sizes[0][0]]).astype(boxes.dtype).reshape(1, 1, 2)
    sizes_wh = jnp.tile(sizes_wh, (1, 1, 2))
    boxes_xy = boxes_xy * sizes_wh
    scores = jax.nn.sigmoid(logits)
    num_top_queries = 300
    num_classes = 80
    flat = scores.reshape(scores.shape[0], -1)
    scores_top, index = jax.lax.top_k(flat, num_top_queries)
    labels = index - (index // num_classes) * num_classes
    qidx = index // num_classes
    gidx = jnp.repeat(qidx[..., None], boxes_xy.shape[-1], axis=-1)
    gathered = jnp.take_along_axis(boxes_xy, gidx, axis=1)
    out = jnp.concatenate([labels[..., None].astype(scores_top.dtype), scores_top[..., None], gathered], axis=-1)
    return out


# speedup vs baseline: 6.5060x; 6.5060x over previous
import functools

import jax
import jax.numpy as jnp
from jax import lax
from jax.experimental import pallas as pl
from jax.experimental.pallas import tpu as pltpu
from jax.experimental.pallas import tpu_sc as plsc

N, Q, K = 16, 5000, 80
QK = Q * K
TOPK = 300
NCORES, NSUB, LANES = 2, 16, 16
ROWS_PER_CORE = N // NCORES
CHUNK = QK // NSUB
NV = CHUNK // LANES
CAP = 128
CAND = NSUB * CAP
TARGET = 330
NBF = 4096
NBC = 256
TOPP = 384
HALF = 152
SENT_IDX = 1 << 22


def _mono_key_u32(d):
    x = lax.bitcast_convert_type(d, jnp.int32)
    flip = lax.shift_right_arithmetic(x, 31)
    key = lax.bitwise_xor(x, lax.bitwise_or(flip, jnp.int32(-(2 ** 31))))
    return lax.bitcast_convert_type(key, jnp.uint32)


def _cand_body(lg_hbm, candv_hbm, candi_hbm,
               data, histf, histc, redc, gtmp, ftmp, fall, cbv, cbi, shared):
    c = lax.axis_index("c")
    s = lax.axis_index("s")
    lane = lax.iota(jnp.int32, LANES)
    ones = jnp.ones((LANES,), jnp.int32)
    zeros = jnp.zeros((LANES,), jnp.int32)

    def _clr_f(i, _):
        histf[pl.ds(i * LANES, LANES)] = zeros
        return 0
    lax.fori_loop(0, NBF * LANES // LANES, _clr_f, 0)

    def _clr_c(i, _):
        histc[pl.ds(i * LANES, LANES)] = zeros
        return 0
    lax.fori_loop(0, NBC * LANES // LANES, _clr_c, 0)

    def _row(rr, _):
        r = c * ROWS_PER_CORE + rr
        base_in = r * QK + s * CHUNK
        pltpu.sync_copy(lg_hbm.at[pl.ds(base_in, CHUNK)],
                        data.at[pl.ds(0, CHUNK)])

        def _p1(i, _):
            d = data[pl.ds(i * LANES, LANES)]
            keyu = _mono_key_u32(d)
            binf = lax.convert_element_type(
                lax.shift_right_logical(keyu, jnp.uint32(20)), jnp.int32)
            binc = lax.shift_right_arithmetic(binf, 4)
            valid = jnp.minimum(jnp.int32(CHUNK) - i * LANES, LANES)
            m = lane < valid
            plsc.addupdate_scatter(histf, [lane * NBF + binf], ones, mask=m)
            plsc.addupdate_scatter(histc, [lane * NBC + binc], ones, mask=m)
            return 0
        lax.fori_loop(0, NV + 1, _p1, 0)

        def _red(g, _):
            acc = zeros
            for t in range(LANES):
                acc = acc + histc[pl.ds(t * NBC + g * LANES, LANES)]
            redc[pl.ds(g * LANES, LANES)] = acc
            return 0
        lax.fori_loop(0, NBC // LANES, _red, 0)
        pltpu.sync_copy(redc, shared.at[pl.ds(s * NBC, NBC)])
        plsc.subcore_barrier()

        pltpu.sync_copy(shared.at[pl.ds(0, NSUB * NBC)], gtmp)

        def _gsum(g, _):
            acc = zeros
            for t in range(NSUB):
                acc = acc + gtmp[pl.ds(t * NBC + g * LANES, LANES)]
            redc[pl.ds(g * LANES, LANES)] = acc
            return 0
        lax.fori_loop(0, NBC // LANES, _gsum, 0)

        def _cscan(gi, carry):
            cum, found, bsel, cum_above = carry
            g = 15 - gi
            v = redc[pl.ds(g * LANES, LANES)]
            vd = lax.rev(v, (0,))
            cd = plsc.cumsum(vd) + cum
            sel = cd >= TARGET
            anyc = jnp.max(plsc.all_reduce_population_count(sel)) > 0
            j = jnp.where(anyc, jnp.max(plsc.all_reduce_ffs(sel)), 0)
            cdj = jnp.max(jnp.where(lane == j, cd, 0))
            vdj = jnp.max(jnp.where(lane == j, vd, 0))
            take = (found == 0) & anyc
            bsel = jnp.where(take, g * LANES + 15 - j, bsel)
            cum_above = jnp.where(take, cdj - vdj, cum_above)
            found = jnp.where(anyc, 1, found)
            return (jnp.max(cd), found, bsel, cum_above)
        _, _, bc, cum_above = lax.fori_loop(
            0, NBC // LANES, _cscan,
            (jnp.int32(0), jnp.int32(0), jnp.int32(0), jnp.int32(0)))

        accf = zeros
        for t in range(LANES):
            accf = accf + histf[pl.ds(t * NBF + bc * LANES, LANES)]
        ftmp[...] = accf
        pltpu.sync_copy(ftmp, shared.at[pl.ds(NSUB * NBC + s * LANES, LANES)])
        plsc.subcore_barrier()
        pltpu.sync_copy(shared.at[pl.ds(NSUB * NBC, NSUB * LANES)], fall)
        accg = zeros
        for t in range(NSUB):
            accg = accg + fall[pl.ds(t * LANES, LANES)]
        vd = lax.rev(accg, (0,))
        cd = plsc.cumsum(vd) + cum_above
        sel = cd >= TARGET
        j = jnp.max(plsc.all_reduce_ffs(sel))
        bf = bc * LANES + 15 - j
        thr = lax.shift_left(lax.convert_element_type(bf, jnp.uint32),
                             jnp.uint32(20))

        neg = jnp.full((LANES,), -1e30, jnp.float32)
        big = jnp.full((LANES,), SENT_IDX, jnp.int32)
        for t in range(CAP // LANES + 1):
            cbv[pl.ds(t * LANES, LANES)] = neg
            cbi[pl.ds(t * LANES, LANES)] = big

        idx_base = s * CHUNK

        def _p2(i, off):
            d = data[pl.ds(i * LANES, LANES)]
            keyu = _mono_key_u32(d)
            binf = lax.convert_element_type(
                lax.shift_right_logical(keyu, jnp.uint32(20)), jnp.int32)
            binc = lax.shift_right_arithmetic(binf, 4)
            valid = jnp.minimum(jnp.int32(CHUNK) - i * LANES, LANES)
            mv = lane < valid
            plsc.store_scatter(histf, [lane * NBF + binf], zeros, mask=mv)
            plsc.store_scatter(histc, [lane * NBC + binc], zeros, mask=mv)
            m = mv & (keyu >= thr) & (off < CAP)
            cnt = jnp.max(plsc.all_reduce_population_count(m))
            plsc.store_compressed(cbv.at[pl.ds(off, LANES)], d, mask=m)
            gi = idx_base + i * LANES + lane
            plsc.store_compressed(cbi.at[pl.ds(off, LANES)], gi, mask=m)
            return off + cnt
        lax.fori_loop(0, NV + 1, _p2, jnp.int32(0))

        base_out = r * CAND + s * CAP
        pltpu.sync_copy(cbv.at[pl.ds(0, CAP)],
                        candv_hbm.at[pl.ds(base_out, CAP)])
        pltpu.sync_copy(cbi.at[pl.ds(0, CAP)],
                        candi_hbm.at[pl.ds(base_out, CAP)])
        return 0
    lax.fori_loop(0, ROWS_PER_CORE, _row, 0)


def _cand_call(lgf):
    mesh = plsc.VectorSubcoreMesh(core_axis_name="c", subcore_axis_name="s")
    fn = pl.kernel(
        _cand_body,
        out_type=[jax.ShapeDtypeStruct((N * CAND,), jnp.float32),
                  jax.ShapeDtypeStruct((N * CAND,), jnp.int32)],
        mesh=mesh,
        scratch_types=[
            pltpu.VMEM((CHUNK + LANES,), jnp.float32),
            pltpu.VMEM((NBF * LANES,), jnp.int32),
            pltpu.VMEM((NBC * LANES,), jnp.int32),
            pltpu.VMEM((NBC,), jnp.int32),
            pltpu.VMEM((NSUB * NBC,), jnp.int32),
            pltpu.VMEM((LANES,), jnp.int32),
            pltpu.VMEM((NSUB * LANES,), jnp.int32),
            pltpu.VMEM((CAP + LANES,), jnp.float32),
            pltpu.VMEM((CAP + LANES,), jnp.int32),
            pltpu.VMEM_SHARED((NSUB * NBC + NSUB * LANES,), jnp.int32),
        ],
        compiler_params=pltpu.CompilerParams(needs_layout_passes=False),
        name="detr_candidates_sc",
    )
    return fn(lgf)


def _sort_body(s_ref, i_ref, os_ref, oi_ref):
    sc = s_ref[...]
    ix = i_ref[...]
    col = lax.broadcasted_iota(jnp.int32, (N, CAND), 1)
    k = 2
    while k <= CAND:
        d = k // 2
        while d >= 1:
            pair_desc = (col & k) == 0
            is_lower = (col & d) == 0
            take_max = pair_desc == is_lower
            sp = jnp.where(is_lower, jnp.roll(sc, -d, axis=1),
                           jnp.roll(sc, d, axis=1))
            ip = jnp.where(is_lower, jnp.roll(ix, -d, axis=1),
                           jnp.roll(ix, d, axis=1))
            greater = (sc > sp) | ((sc == sp) & (ix < ip))
            sel = greater == take_max
            sc = jnp.where(sel, sc, sp)
            ix = jnp.where(sel, ix, ip)
            d //= 2
        k *= 2
    os_ref[...] = sc[:, :TOPP]
    oi_ref[...] = ix[:, :TOPP]


def _sort_call(scores, cidx):
    return pl.pallas_call(
        _sort_body,
        out_shape=[jax.ShapeDtypeStruct((N, TOPP), jnp.float32),
                   jax.ShapeDtypeStruct((N, TOPP), jnp.int32)],
    )(scores, cidx)


def _asm_body(ss_hbm, si_hbm, bx_hbm, wv_hbm, hv_hbm, out_hbm,
              sbuf, ibuf, qbuf, brows, obuf, wbuf, hbuf, sem):
    c = lax.axis_index("c")
    s = lax.axis_index("s")
    lane = lax.iota(jnp.int32, LANES)
    w = s * NCORES + c
    r = w // 2
    h = w % 2
    start = h * HALF

    pltpu.sync_copy(ss_hbm.at[pl.ds(r * TOPP + start, HALF)],
                    sbuf.at[pl.ds(0, HALF)])
    pltpu.sync_copy(si_hbm.at[pl.ds(r * TOPP + start, HALF)],
                    ibuf.at[pl.ds(0, HALF)])
    pltpu.sync_copy(wv_hbm, wbuf)
    pltpu.sync_copy(hv_hbm, hbuf)

    def _qidx(t, _):
        ix = ibuf[pl.ds(t * LANES, LANES)]
        qi = jnp.minimum(ix // K, Q - 1)
        base = (r * Q + qi) * 4
        j = t // 5
        o = (t - j * 5) * LANES
        for cc in range(4):
            qbuf[j * 4 + cc, pl.ds(o, LANES)] = base + cc
        return 0
    lax.fori_loop(0, 10, _qidx, 0)

    for j in range(8):
        pltpu.async_copy(bx_hbm.at[qbuf.at[j]], brows.at[j], sem).wait()

    wsp = wbuf[...]
    hsp = hbuf[...]

    def _asm(t, _):
        jr = t * LANES + lane
        ix = ibuf[pl.ds(t * LANES, LANES)]
        score = sbuf[pl.ds(t * LANES, LANES)]
        label = lax.convert_element_type(ix - (ix // K) * K, jnp.float32)
        jb = t // 5
        o = (t - jb * 5) * LANES
        cx = brows[jb * 4 + 0, pl.ds(o, LANES)]
        cy = brows[jb * 4 + 1, pl.ds(o, LANES)]
        bw = brows[jb * 4 + 2, pl.ds(o, LANES)]
        bh = brows[jb * 4 + 3, pl.ds(o, LANES)]
        x = (cx - bw * 0.5) * wsp
        y = (cy - bh * 0.5) * hsp
        bw = bw * wsp
        bh = bh * hsp
        addr = jr * 6
        plsc.store_scatter(obuf, [addr], label)
        plsc.store_scatter(obuf, [addr + 1], score)
        plsc.store_scatter(obuf, [addr + 2], x)
        plsc.store_scatter(obuf, [addr + 3], y)
        plsc.store_scatter(obuf, [addr + 4], bw)
        plsc.store_scatter(obuf, [addr + 5], bh)
        return 0
    lax.fori_loop(0, 10, _asm, 0)

    base_out = r * (TOPK * 6) + start * 6

    @pl.when(h == 0)
    def _():
        pltpu.sync_copy(obuf.at[pl.ds(0, HALF * 6)],
                        out_hbm.at[pl.ds(base_out, HALF * 6)])

    @pl.when(h == 1)
    def _():
        pltpu.sync_copy(obuf.at[pl.ds(0, (TOPK - HALF) * 6)],
                        out_hbm.at[pl.ds(base_out, (TOPK - HALF) * 6)])


def _asm_call(ss, si, boxes2d, wv, hv):
    mesh = plsc.VectorSubcoreMesh(core_axis_name="c", subcore_axis_name="s")
    fn = pl.kernel(
        _asm_body,
        out_type=jax.ShapeDtypeStruct((N * TOPK * 6,), jnp.float32),
        mesh=mesh,
        scratch_types=[
            pltpu.VMEM((160,), jnp.float32),
            pltpu.VMEM((160,), jnp.int32),
            pltpu.VMEM((8, 80), jnp.int32),
            pltpu.VMEM((8, 80), jnp.float32),
            pltpu.VMEM((160 * 6,), jnp.float32),
            pltpu.VMEM((LANES,), jnp.float32),
            pltpu.VMEM((LANES,), jnp.float32),
            pltpu.SemaphoreType.DMA,
        ],
        compiler_params=pltpu.CompilerParams(needs_layout_passes=False),
        name="detr_assemble_sc",
    )
    return fn(ss, si, boxes2d, wv, hv)


def kernel(logits, boxes, original_sizes):
    lgf = logits.reshape(-1)
    candv, candi = _cand_call(lgf)
    cscores = jax.nn.sigmoid(candv).reshape(N, CAND)
    ss, si = _sort_call(cscores, candi.reshape(N, CAND))
    wsc = original_sizes[0, 1].astype(jnp.float32)
    hsc = original_sizes[0, 0].astype(jnp.float32)
    wv = jnp.full((LANES,), 1.0, jnp.float32) * wsc
    hv = jnp.full((LANES,), 1.0, jnp.float32) * hsc
    out = _asm_call(ss.reshape(-1), si.reshape(-1),
                    boxes.reshape(-1), wv, hv)
    return out.reshape(N, TOPK, 6)


# speedup vs baseline: 8.8831x; 1.3654x over previous
import functools

import jax
import jax.numpy as jnp
from jax import lax
from jax.experimental import pallas as pl
from jax.experimental.pallas import tpu as pltpu
from jax.experimental.pallas import tpu_sc as plsc

N, Q, K = 16, 5000, 80
QK = Q * K
TOPK = 300
NCORES, NSUB, LANES = 2, 16, 16
ROWS_PER_CORE = N // NCORES
CHUNK = QK // NSUB
NV = CHUNK // LANES
UNROLL = 8
NITER = -(-CHUNK // (UNROLL * LANES))
NFULL = CHUNK // (UNROLL * LANES)
NTAIL = -(-(CHUNK - NFULL * UNROLL * LANES) // LANES)
UNROLL3 = 16
NITER3 = -(-CHUNK // (UNROLL3 * LANES))
CAP = 128
CAND = NSUB * CAP
TARGET = 330
NBF = 4096
NBC = 256
TOPP = 384
HALF = 152
SENT_IDX = 1 << 22


def _mono_key_u32(d):
    x = lax.bitcast_convert_type(d, jnp.int32)
    flip = lax.shift_right_arithmetic(x, 31)
    key = lax.bitwise_xor(x, lax.bitwise_or(flip, jnp.int32(-(2 ** 31))))
    return lax.bitcast_convert_type(key, jnp.uint32)


def _cand_body(lg_hbm, candv_hbm, candi_hbm,
               data, histf, histc, redc, gtmp, ftmp, fall, cbv, cbi, offbuf,
               shared):
    c = lax.axis_index("c")
    s = lax.axis_index("s")
    lane = lax.iota(jnp.int32, LANES)
    ones = jnp.ones((LANES,), jnp.int32)
    zeros = jnp.zeros((LANES,), jnp.int32)

    def _clr_f(i, _):
        histf[pl.ds(i * LANES, LANES)] = zeros
        return 0
    lax.fori_loop(0, NBF * LANES // LANES, _clr_f, 0)

    def _clr_c(i, _):
        histc[pl.ds(i * LANES, LANES)] = zeros
        return 0
    lax.fori_loop(0, NBC * LANES // LANES, _clr_c, 0)

    def _row(rr, _):
        r = c * ROWS_PER_CORE + rr
        base_in = r * QK + s * CHUNK
        pltpu.sync_copy(lg_hbm.at[pl.ds(base_in, CHUNK)],
                        data.at[pl.ds(0, CHUNK)])

        lanef = lane * NBF
        lanec = lane * NBC

        def _p1(i, _):
            for j in range(UNROLL):
                o = i * (UNROLL * LANES) + j * LANES
                d = data[pl.ds(o, LANES)]
                keyu = _mono_key_u32(d)
                binf = lax.convert_element_type(
                    lax.shift_right_logical(keyu, jnp.uint32(20)), jnp.int32)
                binc = lax.shift_right_arithmetic(binf, 4)
                plsc.addupdate_scatter(histf, [lanef + binf], ones)
                plsc.addupdate_scatter(histc, [lanec + binc], ones)
            return 0
        lax.fori_loop(0, NFULL, _p1, 0)
        for j in range(NTAIL):
            o = NFULL * UNROLL * LANES + j * LANES
            d = data[pl.ds(o, LANES)]
            keyu = _mono_key_u32(d)
            binf = lax.convert_element_type(
                lax.shift_right_logical(keyu, jnp.uint32(20)), jnp.int32)
            binc = lax.shift_right_arithmetic(binf, 4)
            m = lane < (jnp.int32(CHUNK) - o)
            plsc.addupdate_scatter(histf, [lanef + binf], ones, mask=m)
            plsc.addupdate_scatter(histc, [lanec + binc], ones, mask=m)

        def _red(g, _):
            acc = zeros
            for t in range(LANES):
                acc = acc + histc[pl.ds(t * NBC + g * LANES, LANES)]
            redc[pl.ds(g * LANES, LANES)] = acc
            return 0
        lax.fori_loop(0, NBC // LANES, _red, 0)
        pltpu.sync_copy(redc, shared.at[pl.ds(s * NBC, NBC)])
        plsc.subcore_barrier()

        pltpu.sync_copy(shared.at[pl.ds(0, NSUB * NBC)], gtmp)

        def _gsum(g, _):
            acc = zeros
            for t in range(NSUB):
                acc = acc + gtmp[pl.ds(t * NBC + g * LANES, LANES)]
            redc[pl.ds(g * LANES, LANES)] = acc
            return 0
        lax.fori_loop(0, NBC // LANES, _gsum, 0)

        def _cscan(gi, carry):
            cum, found, bsel, cum_above = carry
            g = 15 - gi
            v = redc[pl.ds(g * LANES, LANES)]
            vd = lax.rev(v, (0,))
            cd = plsc.cumsum(vd) + cum
            sel = cd >= TARGET
            anyc = jnp.max(plsc.all_reduce_population_count(sel)) > 0
            j = jnp.where(anyc, jnp.max(plsc.all_reduce_ffs(sel)), 0)
            cdj = jnp.max(jnp.where(lane == j, cd, 0))
            vdj = jnp.max(jnp.where(lane == j, vd, 0))
            take = (found == 0) & anyc
            bsel = jnp.where(take, g * LANES + 15 - j, bsel)
            cum_above = jnp.where(take, cdj - vdj, cum_above)
            found = jnp.where(anyc, 1, found)
            return (jnp.max(cd), found, bsel, cum_above)
        _, _, bc, cum_above = lax.fori_loop(
            0, NBC // LANES, _cscan,
            (jnp.int32(0), jnp.int32(0), jnp.int32(0), jnp.int32(0)))

        accf = zeros
        for t in range(LANES):
            accf = accf + histf[pl.ds(t * NBF + bc * LANES, LANES)]
        ftmp[...] = accf
        pltpu.sync_copy(ftmp, shared.at[pl.ds(NSUB * NBC + s * LANES, LANES)])
        plsc.subcore_barrier()
        pltpu.sync_copy(shared.at[pl.ds(NSUB * NBC, NSUB * LANES)], fall)
        accg = zeros
        for t in range(NSUB):
            accg = accg + fall[pl.ds(t * LANES, LANES)]
        vd = lax.rev(accg, (0,))
        cd = plsc.cumsum(vd) + cum_above
        sel = cd >= TARGET
        j = jnp.max(plsc.all_reduce_ffs(sel))
        bf = bc * LANES + 15 - j
        thr = lax.shift_left(lax.convert_element_type(bf, jnp.uint32),
                             jnp.uint32(20))

        def _z1(i, _):
            for j in range(UNROLL):
                histf[pl.ds(i * (UNROLL * LANES) + j * LANES, LANES)] = zeros
            return 0
        lax.fori_loop(0, NBF * LANES // (UNROLL * LANES), _z1, 0)

        def _z2(i, _):
            for j in range(UNROLL):
                histc[pl.ds(i * (UNROLL * LANES) + j * LANES, LANES)] = zeros
            return 0
        lax.fori_loop(0, NBC * LANES // (UNROLL * LANES), _z2, 0)

        neg = jnp.full((LANES,), -1e30, jnp.float32)
        big = jnp.full((LANES,), SENT_IDX, jnp.int32)
        for t in range(CAP // LANES + 1):
            cbv[pl.ds(t * LANES, LANES)] = neg
            cbi[pl.ds(t * LANES, LANES)] = big
        offbuf[...] = zeros

        idx0 = s * CHUNK + lane
        capv = jnp.full((LANES,), CAP + LANES - 1, jnp.int32)

        def _pc(i, _):
            ms, dsv = [], []
            for j in range(UNROLL3):
                o = i * (UNROLL3 * LANES) + j * LANES
                d = data[pl.ds(o, LANES)]
                keyu = _mono_key_u32(d)
                ms.append((keyu >= thr) & (lane < (jnp.int32(CHUNK) - o)))
                dsv.append(d)
            om = ms[0]
            for j in range(1, UNROLL3):
                om = om | ms[j]
            anyc = jnp.max(plsc.all_reduce_population_count(om)) > 0

            @pl.when(anyc)
            def _():
                off = offbuf[...]
                poss, offs = [], []
                for j in range(UNROLL3):
                    poss.append(plsc.cumsum(
                        lax.convert_element_type(ms[j], jnp.int32)))
                    offs.append(off)
                    off = off + plsc.all_reduce_population_count(ms[j])
                for j in range(UNROLL3):
                    o = i * (UNROLL3 * LANES) + j * LANES
                    addr = jnp.minimum(offs[j] + poss[j] - 1, capv)
                    plsc.store_scatter(cbv, [addr], dsv[j], mask=ms[j])
                    plsc.store_scatter(cbi, [addr], idx0 + o, mask=ms[j])
                offbuf[...] = off
            return 0
        lax.fori_loop(0, NITER3, _pc, 0)

        base_out = r * CAND + s * CAP
        pltpu.sync_copy(cbv.at[pl.ds(0, CAP)],
                        candv_hbm.at[pl.ds(base_out, CAP)])
        pltpu.sync_copy(cbi.at[pl.ds(0, CAP)],
                        candi_hbm.at[pl.ds(base_out, CAP)])
        return 0
    lax.fori_loop(0, ROWS_PER_CORE, _row, 0)


def _cand_call(lgf):
    mesh = plsc.VectorSubcoreMesh(core_axis_name="c", subcore_axis_name="s")
    fn = pl.kernel(
        _cand_body,
        out_type=[jax.ShapeDtypeStruct((N * CAND,), jnp.float32),
                  jax.ShapeDtypeStruct((N * CAND,), jnp.int32)],
        mesh=mesh,
        scratch_types=[
            pltpu.VMEM((NITER * UNROLL * LANES,), jnp.float32),
            pltpu.VMEM((NBF * LANES,), jnp.int32),
            pltpu.VMEM((NBC * LANES,), jnp.int32),
            pltpu.VMEM((NBC,), jnp.int32),
            pltpu.VMEM((NSUB * NBC,), jnp.int32),
            pltpu.VMEM((LANES,), jnp.int32),
            pltpu.VMEM((NSUB * LANES,), jnp.int32),
            pltpu.VMEM((CAP + LANES,), jnp.float32),
            pltpu.VMEM((CAP + LANES,), jnp.int32),
            pltpu.VMEM((LANES,), jnp.int32),
            pltpu.VMEM_SHARED((NSUB * NBC + NSUB * LANES,), jnp.int32),
        ],
        compiler_params=pltpu.CompilerParams(needs_layout_passes=False),
        name="detr_candidates_sc",
    )
    return fn(lgf)


def _sort_body(s_ref, i_ref, os_ref, oi_ref):
    sc = s_ref[...]
    ix = i_ref[...]
    col = lax.broadcasted_iota(jnp.int32, (N, CAND), 1)
    k = 2
    while k <= CAND:
        d = k // 2
        while d >= 1:
            pair_desc = (col & k) == 0
            is_lower = (col & d) == 0
            take_max = pair_desc == is_lower
            sp = jnp.where(is_lower, jnp.roll(sc, -d, axis=1),
                           jnp.roll(sc, d, axis=1))
            ip = jnp.where(is_lower, jnp.roll(ix, -d, axis=1),
                           jnp.roll(ix, d, axis=1))
            greater = (sc > sp) | ((sc == sp) & (ix < ip))
            sel = greater == take_max
            sc = jnp.where(sel, sc, sp)
            ix = jnp.where(sel, ix, ip)
            d //= 2
        k *= 2
    os_ref[...] = sc[:, :TOPP]
    oi_ref[...] = ix[:, :TOPP]


def _sort_call(scores, cidx):
    return pl.pallas_call(
        _sort_body,
        out_shape=[jax.ShapeDtypeStruct((N, TOPP), jnp.float32),
                   jax.ShapeDtypeStruct((N, TOPP), jnp.int32)],
    )(scores, cidx)


def _asm_body(ss_hbm, si_hbm, bx_hbm, wv_hbm, hv_hbm, out_hbm,
              sbuf, ibuf, qbuf, brows, obuf, wbuf, hbuf, sem):
    c = lax.axis_index("c")
    s = lax.axis_index("s")
    lane = lax.iota(jnp.int32, LANES)
    w = s * NCORES + c
    r = w // 2
    h = w % 2
    start = h * HALF

    pltpu.sync_copy(ss_hbm.at[pl.ds(r * TOPP + start, HALF)],
                    sbuf.at[pl.ds(0, HALF)])
    pltpu.sync_copy(si_hbm.at[pl.ds(r * TOPP + start, HALF)],
                    ibuf.at[pl.ds(0, HALF)])
    pltpu.sync_copy(wv_hbm, wbuf)
    pltpu.sync_copy(hv_hbm, hbuf)

    def _qidx(t, _):
        ix = ibuf[pl.ds(t * LANES, LANES)]
        qi = jnp.minimum(ix // K, Q - 1)
        base = (r * Q + qi) * 4
        j = t // 5
        o = (t - j * 5) * LANES
        for cc in range(4):
            qbuf[j * 4 + cc, pl.ds(o, LANES)] = base + cc
        return 0
    lax.fori_loop(0, 10, _qidx, 0)

    for j in range(8):
        pltpu.async_copy(bx_hbm.at[qbuf.at[j]], brows.at[j], sem).wait()

    wsp = wbuf[...]
    hsp = hbuf[...]

    def _asm(t, _):
        jr = t * LANES + lane
        ix = ibuf[pl.ds(t * LANES, LANES)]
        score = sbuf[pl.ds(t * LANES, LANES)]
        label = lax.convert_element_type(ix - (ix // K) * K, jnp.float32)
        jb = t // 5
        o = (t - jb * 5) * LANES
        cx = brows[jb * 4 + 0, pl.ds(o, LANES)]
        cy = brows[jb * 4 + 1, pl.ds(o, LANES)]
        bw = brows[jb * 4 + 2, pl.ds(o, LANES)]
        bh = brows[jb * 4 + 3, pl.ds(o, LANES)]
        x = (cx - bw * 0.5) * wsp
        y = (cy - bh * 0.5) * hsp
        bw = bw * wsp
        bh = bh * hsp
        addr = jr * 6
        plsc.store_scatter(obuf, [addr], label)
        plsc.store_scatter(obuf, [addr + 1], score)
        plsc.store_scatter(obuf, [addr + 2], x)
        plsc.store_scatter(obuf, [addr + 3], y)
        plsc.store_scatter(obuf, [addr + 4], bw)
        plsc.store_scatter(obuf, [addr + 5], bh)
        return 0
    lax.fori_loop(0, 10, _asm, 0)

    base_out = r * (TOPK * 6) + start * 6

    @pl.when(h == 0)
    def _():
        pltpu.sync_copy(obuf.at[pl.ds(0, HALF * 6)],
                        out_hbm.at[pl.ds(base_out, HALF * 6)])

    @pl.when(h == 1)
    def _():
        pltpu.sync_copy(obuf.at[pl.ds(0, (TOPK - HALF) * 6)],
                        out_hbm.at[pl.ds(base_out, (TOPK - HALF) * 6)])


def _asm_call(ss, si, boxes2d, wv, hv):
    mesh = plsc.VectorSubcoreMesh(core_axis_name="c", subcore_axis_name="s")
    fn = pl.kernel(
        _asm_body,
        out_type=jax.ShapeDtypeStruct((N * TOPK * 6,), jnp.float32),
        mesh=mesh,
        scratch_types=[
            pltpu.VMEM((160,), jnp.float32),
            pltpu.VMEM((160,), jnp.int32),
            pltpu.VMEM((8, 80), jnp.int32),
            pltpu.VMEM((8, 80), jnp.float32),
            pltpu.VMEM((160 * 6,), jnp.float32),
            pltpu.VMEM((LANES,), jnp.float32),
            pltpu.VMEM((LANES,), jnp.float32),
            pltpu.SemaphoreType.DMA,
        ],
        compiler_params=pltpu.CompilerParams(needs_layout_passes=False),
        name="detr_assemble_sc",
    )
    return fn(ss, si, boxes2d, wv, hv)


def kernel(logits, boxes, original_sizes):
    lgf = logits.reshape(-1)
    candv, candi = _cand_call(lgf)
    cscores = jax.nn.sigmoid(candv).reshape(N, CAND)
    ss, si = _sort_call(cscores, candi.reshape(N, CAND))
    wsc = original_sizes[0, 1].astype(jnp.float32)
    hsc = original_sizes[0, 0].astype(jnp.float32)
    wv = jnp.full((LANES,), 1.0, jnp.float32) * wsc
    hv = jnp.full((LANES,), 1.0, jnp.float32) * hsc
    out = _asm_call(ss.reshape(-1), si.reshape(-1),
                    boxes.reshape(-1), wv, hv)
    return out.reshape(N, TOPK, 6)


# speedup vs baseline: 9.4638x; 1.0654x over previous
import functools

import jax
import jax.numpy as jnp
from jax import lax
from jax.experimental import pallas as pl
from jax.experimental.pallas import tpu as pltpu
from jax.experimental.pallas import tpu_sc as plsc

N, Q, K = 16, 5000, 80
QK = Q * K
TOPK = 300
NCORES, NSUB, LANES = 2, 16, 16
ROWS_PER_CORE = N // NCORES
CHUNK = QK // NSUB
NV = CHUNK // LANES
UNROLL = 8
NITER = -(-CHUNK // (UNROLL * LANES))
NFULL = CHUNK // (UNROLL * LANES)
NTAIL = -(-(CHUNK - NFULL * UNROLL * LANES) // LANES)
UNROLL3 = 16
NITER3 = -(-CHUNK // (UNROLL3 * LANES))
CAP = 128
CAND = NSUB * CAP
TARGET = 330
NBF = 4096
NBC = 256
TOPP = 384
HALF = 152
SENT_IDX = 1 << 22


def _mono_key_u32(d):
    x = lax.bitcast_convert_type(d, jnp.int32)
    flip = lax.shift_right_arithmetic(x, 31)
    key = lax.bitwise_xor(x, lax.bitwise_or(flip, jnp.int32(-(2 ** 31))))
    return lax.bitcast_convert_type(key, jnp.uint32)


def _cand_body(lg_hbm, candv_hbm, candi_hbm,
               data, histf, histc, histt, ftmpt, redc, gtmp, ftmp, fall,
               cbv, cbi, offbuf, shared):
    c = lax.axis_index("c")
    s = lax.axis_index("s")
    lane = lax.iota(jnp.int32, LANES)
    ones = jnp.ones((LANES,), jnp.int32)
    zeros = jnp.zeros((LANES,), jnp.int32)

    def _clr_f(i, _):
        histf[pl.ds(i * LANES, LANES)] = zeros
        return 0
    lax.fori_loop(0, NBF * LANES // LANES, _clr_f, 0)

    def _clr_c(i, _):
        histc[pl.ds(i * LANES, LANES)] = zeros
        return 0
    lax.fori_loop(0, NBC * LANES // LANES, _clr_c, 0)

    def _row(rr, _):
        r = c * ROWS_PER_CORE + rr
        base_in = r * QK + s * CHUNK
        pltpu.sync_copy(lg_hbm.at[pl.ds(base_in, CHUNK)],
                        data.at[pl.ds(0, CHUNK)])

        def _haddr(keyu):
            b16 = lax.convert_element_type(
                lax.shift_right_logical(keyu, jnp.uint32(16)), jnp.int32)
            af = lax.bitwise_or(lax.bitwise_and(b16, jnp.int32(0xFFF0)), lane)
            ac = lax.bitwise_or(
                lax.bitwise_and(lax.shift_right_arithmetic(b16, 4),
                                jnp.int32(0xFF0)), lane)
            return af, ac

        def _p1(i, _):
            for j in range(UNROLL):
                o = i * (UNROLL * LANES) + j * LANES
                d = data[pl.ds(o, LANES)]
                af, ac = _haddr(_mono_key_u32(d))
                plsc.addupdate_scatter(histf, [af], ones)
                plsc.addupdate_scatter(histc, [ac], ones)
            return 0
        lax.fori_loop(0, NFULL, _p1, 0)
        for j in range(NTAIL):
            o = NFULL * UNROLL * LANES + j * LANES
            d = data[pl.ds(o, LANES)]
            af, ac = _haddr(_mono_key_u32(d))
            m = lane < (jnp.int32(CHUNK) - o)
            plsc.addupdate_scatter(histf, [af], ones, mask=m)
            plsc.addupdate_scatter(histc, [ac], ones, mask=m)

        def _tr(b, _):
            v = histc[pl.ds(b * LANES, LANES)]
            plsc.store_scatter(histt, [lane * (NBC + 1) + b], v)
            return 0
        lax.fori_loop(0, NBC, _tr, 0)

        def _red(g, _):
            acc = zeros
            for t in range(LANES):
                acc = acc + histt[pl.ds(t * (NBC + 1) + g * LANES, LANES)]
            redc[pl.ds(g * LANES, LANES)] = acc
            return 0
        lax.fori_loop(0, NBC // LANES, _red, 0)
        pltpu.sync_copy(redc, shared.at[pl.ds(s * NBC, NBC)])
        plsc.subcore_barrier()

        pltpu.sync_copy(shared.at[pl.ds(0, NSUB * NBC)], gtmp)

        def _gsum(g, _):
            acc = zeros
            for t in range(NSUB):
                acc = acc + gtmp[pl.ds(t * NBC + g * LANES, LANES)]
            redc[pl.ds(g * LANES, LANES)] = acc
            return 0
        lax.fori_loop(0, NBC // LANES, _gsum, 0)

        def _cscan(gi, carry):
            cum, found, bsel, cum_above = carry
            g = 15 - gi
            v = redc[pl.ds(g * LANES, LANES)]
            vd = lax.rev(v, (0,))
            cd = plsc.cumsum(vd) + cum
            sel = cd >= TARGET
            anyc = jnp.max(plsc.all_reduce_population_count(sel)) > 0
            j = jnp.where(anyc, jnp.max(plsc.all_reduce_ffs(sel)), 0)
            cdj = jnp.max(jnp.where(lane == j, cd, 0))
            vdj = jnp.max(jnp.where(lane == j, vd, 0))
            take = (found == 0) & anyc
            bsel = jnp.where(take, g * LANES + 15 - j, bsel)
            cum_above = jnp.where(take, cdj - vdj, cum_above)
            found = jnp.where(anyc, 1, found)
            return (jnp.max(cd), found, bsel, cum_above)
        _, _, bc, cum_above = lax.fori_loop(
            0, NBC // LANES, _cscan,
            (jnp.int32(0), jnp.int32(0), jnp.int32(0), jnp.int32(0)))

        for k in range(LANES):
            v = histf[pl.ds((bc * LANES + k) * LANES, LANES)]
            plsc.store_scatter(ftmpt, [lane * (LANES + 1) + k], v)
        accf = zeros
        for t in range(LANES):
            accf = accf + ftmpt[pl.ds(t * (LANES + 1), LANES)]
        ftmp[...] = accf
        pltpu.sync_copy(ftmp, shared.at[pl.ds(NSUB * NBC + s * LANES, LANES)])
        plsc.subcore_barrier()
        pltpu.sync_copy(shared.at[pl.ds(NSUB * NBC, NSUB * LANES)], fall)
        accg = zeros
        for t in range(NSUB):
            accg = accg + fall[pl.ds(t * LANES, LANES)]
        vd = lax.rev(accg, (0,))
        cd = plsc.cumsum(vd) + cum_above
        sel = cd >= TARGET
        j = jnp.max(plsc.all_reduce_ffs(sel))
        bf = bc * LANES + 15 - j
        thr = lax.shift_left(lax.convert_element_type(bf, jnp.uint32),
                             jnp.uint32(20))

        def _z1(i, _):
            for j in range(UNROLL):
                histf[pl.ds(i * (UNROLL * LANES) + j * LANES, LANES)] = zeros
            return 0
        lax.fori_loop(0, NBF * LANES // (UNROLL * LANES), _z1, 0)

        def _z2(i, _):
            for j in range(UNROLL):
                histc[pl.ds(i * (UNROLL * LANES) + j * LANES, LANES)] = zeros
            return 0
        lax.fori_loop(0, NBC * LANES // (UNROLL * LANES), _z2, 0)

        neg = jnp.full((LANES,), -1e30, jnp.float32)
        big = jnp.full((LANES,), SENT_IDX, jnp.int32)
        for t in range(CAP // LANES + 1):
            cbv[pl.ds(t * LANES, LANES)] = neg
            cbi[pl.ds(t * LANES, LANES)] = big
        offbuf[...] = zeros

        idx0 = s * CHUNK + lane
        capv = jnp.full((LANES,), CAP + LANES - 1, jnp.int32)

        def _pc(i, _):
            ms, dsv = [], []
            for j in range(UNROLL3):
                o = i * (UNROLL3 * LANES) + j * LANES
                d = data[pl.ds(o, LANES)]
                keyu = _mono_key_u32(d)
                ms.append((keyu >= thr) & (lane < (jnp.int32(CHUNK) - o)))
                dsv.append(d)
            om = ms[0]
            for j in range(1, UNROLL3):
                om = om | ms[j]
            anyc = jnp.max(plsc.all_reduce_population_count(om)) > 0

            @pl.when(anyc)
            def _():
                off = offbuf[...]
                poss, offs = [], []
                for j in range(UNROLL3):
                    poss.append(plsc.cumsum(
                        lax.convert_element_type(ms[j], jnp.int32)))
                    offs.append(off)
                    off = off + plsc.all_reduce_population_count(ms[j])
                for j in range(UNROLL3):
                    o = i * (UNROLL3 * LANES) + j * LANES
                    addr = jnp.minimum(offs[j] + poss[j] - 1, capv)
                    plsc.store_scatter(cbv, [addr], dsv[j], mask=ms[j])
                    plsc.store_scatter(cbi, [addr], idx0 + o, mask=ms[j])
                offbuf[...] = off
            return 0
        lax.fori_loop(0, NITER3, _pc, 0)

        base_out = r * CAND + s * CAP
        pltpu.sync_copy(cbv.at[pl.ds(0, CAP)],
                        candv_hbm.at[pl.ds(base_out, CAP)])
        pltpu.sync_copy(cbi.at[pl.ds(0, CAP)],
                        candi_hbm.at[pl.ds(base_out, CAP)])
        return 0
    lax.fori_loop(0, ROWS_PER_CORE, _row, 0)


def _cand_call(lgf):
    mesh = plsc.VectorSubcoreMesh(core_axis_name="c", subcore_axis_name="s")
    fn = pl.kernel(
        _cand_body,
        out_type=[jax.ShapeDtypeStruct((N * CAND,), jnp.float32),
                  jax.ShapeDtypeStruct((N * CAND,), jnp.int32)],
        mesh=mesh,
        scratch_types=[
            pltpu.VMEM((NITER * UNROLL * LANES,), jnp.float32),
            pltpu.VMEM((NBF * LANES,), jnp.int32),
            pltpu.VMEM((NBC * LANES,), jnp.int32),
            pltpu.VMEM((LANES * (NBC + 1),), jnp.int32),
            pltpu.VMEM((LANES * (LANES + 1) + LANES,), jnp.int32),
            pltpu.VMEM((NBC,), jnp.int32),
            pltpu.VMEM((NSUB * NBC,), jnp.int32),
            pltpu.VMEM((LANES,), jnp.int32),
            pltpu.VMEM((NSUB * LANES,), jnp.int32),
            pltpu.VMEM((CAP + LANES,), jnp.float32),
            pltpu.VMEM((CAP + LANES,), jnp.int32),
            pltpu.VMEM((LANES,), jnp.int32),
            pltpu.VMEM_SHARED((NSUB * NBC + NSUB * LANES,), jnp.int32),
        ],
        compiler_params=pltpu.CompilerParams(needs_layout_passes=False),
        name="detr_candidates_sc",
    )
    return fn(lgf)


def _sort_body(s_ref, i_ref, os_ref, oi_ref):
    sc = s_ref[...]
    ix = i_ref[...]
    col = lax.broadcasted_iota(jnp.int32, (N, CAND), 1)
    k = 2
    while k <= CAND:
        d = k // 2
        while d >= 1:
            pair_desc = (col & k) == 0
            is_lower = (col & d) == 0
            take_max = pair_desc == is_lower
            sp = jnp.where(is_lower, jnp.roll(sc, -d, axis=1),
                           jnp.roll(sc, d, axis=1))
            ip = jnp.where(is_lower, jnp.roll(ix, -d, axis=1),
                           jnp.roll(ix, d, axis=1))
            greater = (sc > sp) | ((sc == sp) & (ix < ip))
            sel = greater == take_max
            sc = jnp.where(sel, sc, sp)
            ix = jnp.where(sel, ix, ip)
            d //= 2
        k *= 2
    os_ref[...] = sc[:, :TOPP]
    oi_ref[...] = ix[:, :TOPP]


def _sort_call(scores, cidx):
    return pl.pallas_call(
        _sort_body,
        out_shape=[jax.ShapeDtypeStruct((N, TOPP), jnp.float32),
                   jax.ShapeDtypeStruct((N, TOPP), jnp.int32)],
    )(scores, cidx)


def _asm_body(ss_hbm, si_hbm, bx_hbm, wv_hbm, hv_hbm, out_hbm,
              sbuf, ibuf, qbuf, brows, obuf, wbuf, hbuf, sem):
    c = lax.axis_index("c")
    s = lax.axis_index("s")
    lane = lax.iota(jnp.int32, LANES)
    w = s * NCORES + c
    r = w // 2
    h = w % 2
    start = h * HALF

    pltpu.sync_copy(ss_hbm.at[pl.ds(r * TOPP + start, HALF)],
                    sbuf.at[pl.ds(0, HALF)])
    pltpu.sync_copy(si_hbm.at[pl.ds(r * TOPP + start, HALF)],
                    ibuf.at[pl.ds(0, HALF)])
    pltpu.sync_copy(wv_hbm, wbuf)
    pltpu.sync_copy(hv_hbm, hbuf)

    def _qidx(t, _):
        ix = ibuf[pl.ds(t * LANES, LANES)]
        qi = jnp.minimum(ix // K, Q - 1)
        base = (r * Q + qi) * 4
        j = t // 5
        o = (t - j * 5) * LANES
        for cc in range(4):
            qbuf[j * 4 + cc, pl.ds(o, LANES)] = base + cc
        return 0
    lax.fori_loop(0, 10, _qidx, 0)

    for j in range(8):
        pltpu.async_copy(bx_hbm.at[qbuf.at[j]], brows.at[j], sem).wait()

    wsp = wbuf[...]
    hsp = hbuf[...]

    def _asm(t, _):
        jr = t * LANES + lane
        ix = ibuf[pl.ds(t * LANES, LANES)]
        score = sbuf[pl.ds(t * LANES, LANES)]
        label = lax.convert_element_type(ix - (ix // K) * K, jnp.float32)
        jb = t // 5
        o = (t - jb * 5) * LANES
        cx = brows[jb * 4 + 0, pl.ds(o, LANES)]
        cy = brows[jb * 4 + 1, pl.ds(o, LANES)]
        bw = brows[jb * 4 + 2, pl.ds(o, LANES)]
        bh = brows[jb * 4 + 3, pl.ds(o, LANES)]
        x = (cx - bw * 0.5) * wsp
        y = (cy - bh * 0.5) * hsp
        bw = bw * wsp
        bh = bh * hsp
        addr = jr * 6
        plsc.store_scatter(obuf, [addr], label)
        plsc.store_scatter(obuf, [addr + 1], score)
        plsc.store_scatter(obuf, [addr + 2], x)
        plsc.store_scatter(obuf, [addr + 3], y)
        plsc.store_scatter(obuf, [addr + 4], bw)
        plsc.store_scatter(obuf, [addr + 5], bh)
        return 0
    lax.fori_loop(0, 10, _asm, 0)

    base_out = r * (TOPK * 6) + start * 6

    @pl.when(h == 0)
    def _():
        pltpu.sync_copy(obuf.at[pl.ds(0, HALF * 6)],
                        out_hbm.at[pl.ds(base_out, HALF * 6)])

    @pl.when(h == 1)
    def _():
        pltpu.sync_copy(obuf.at[pl.ds(0, (TOPK - HALF) * 6)],
                        out_hbm.at[pl.ds(base_out, (TOPK - HALF) * 6)])


def _asm_call(ss, si, boxes2d, wv, hv):
    mesh = plsc.VectorSubcoreMesh(core_axis_name="c", subcore_axis_name="s")
    fn = pl.kernel(
        _asm_body,
        out_type=jax.ShapeDtypeStruct((N * TOPK * 6,), jnp.float32),
        mesh=mesh,
        scratch_types=[
            pltpu.VMEM((160,), jnp.float32),
            pltpu.VMEM((160,), jnp.int32),
            pltpu.VMEM((8, 80), jnp.int32),
            pltpu.VMEM((8, 80), jnp.float32),
            pltpu.VMEM((160 * 6,), jnp.float32),
            pltpu.VMEM((LANES,), jnp.float32),
            pltpu.VMEM((LANES,), jnp.float32),
            pltpu.SemaphoreType.DMA,
        ],
        compiler_params=pltpu.CompilerParams(needs_layout_passes=False),
        name="detr_assemble_sc",
    )
    return fn(ss, si, boxes2d, wv, hv)


def kernel(logits, boxes, original_sizes):
    lgf = logits.reshape(-1)
    candv, candi = _cand_call(lgf)
    cscores = jax.nn.sigmoid(candv).reshape(N, CAND)
    ss, si = _sort_call(cscores, candi.reshape(N, CAND))
    wsc = original_sizes[0, 1].astype(jnp.float32)
    hsc = original_sizes[0, 0].astype(jnp.float32)
    wv = jnp.full((LANES,), 1.0, jnp.float32) * wsc
    hv = jnp.full((LANES,), 1.0, jnp.float32) * hsc
    out = _asm_call(ss.reshape(-1), si.reshape(-1),
                    boxes.reshape(-1), wv, hv)
    return out.reshape(N, TOPK, 6)
